# Initial kernel scaffold; baseline (speedup 1.0000x reference)
import jax
import jax.numpy as jnp
from jax.experimental import pallas as pl


def kernel(x, edge_index, edge_attr, batch, W_in, b_in, W_gcn, b_gcn, bn_gamma, bn_beta, W1, b1, W2, b2, W3, b3):
    raise NotImplementedError("write your pallas kernel here")


# speedup vs baseline: 10.8646x; 10.8646x over previous
import functools

import jax
import jax.numpy as jnp
from jax import lax
from jax.experimental import pallas as pl
from jax.experimental.pallas import tpu as pltpu
from jax.experimental.pallas import tpu_sc as plsc

N = 10000
N_PAD = 10240
NB = 640
GRID = N_PAD // NB
E = 320000
E_PAD = 323584
CHUNK = 128
NCH = E_PAD // 16 // CHUNK
NCH_DEG = E_PAD // 32 // CHUNK
HH = 128
NGRP = 64

_MESH = plsc.VectorSubcoreMesh(core_axis_name="c", subcore_axis_name="s")


@functools.partial(
    pl.kernel,
    out_type=jax.ShapeDtypeStruct((2, N_PAD), jnp.float32),
    mesh=_MESH,
    scratch_types=[
        pltpu.VMEM((NCH_DEG, CHUNK), jnp.int32),
        pltpu.VMEM((NCH_DEG, CHUNK), jnp.float32),
        pltpu.VMEM((NB,), jnp.float32),
        pltpu.VMEM_SHARED((N_PAD,), jnp.float32),
    ],
)
def _sc_deg(col_hbm, ew_hbm, degp_hbm, colv, ewv, zv, acc):
    c = lax.axis_index("c")
    s = lax.axis_index("s")
    wid = c * 16 + s
    for j in range(NB // 16):
        zv[pl.ds(j * 16, 16)] = jnp.zeros((16,), jnp.float32)
    pltpu.sync_copy(zv, acc.at[pl.ds(s * NB, NB)])
    plsc.subcore_barrier()
    pltpu.sync_copy(col_hbm.at[wid], colv)
    pltpu.sync_copy(ew_hbm.at[wid], ewv)

    def body(k, carry):
        pltpu.sync_copy(ewv.at[k], acc.at[colv.at[k]], add=True)
        return carry

    lax.fori_loop(0, NCH_DEG, body, 0)
    plsc.subcore_barrier()
    pltpu.sync_copy(acc.at[pl.ds(s * NB, NB)], degp_hbm.at[c, pl.ds(s * NB, NB)])


@functools.partial(
    pl.kernel,
    out_type=jax.ShapeDtypeStruct((2, N_PAD, HH), jnp.float32),
    mesh=_MESH,
    scratch_types=[
        pltpu.VMEM((2, 2, CHUNK), jnp.int32),
        pltpu.VMEM((2, CHUNK), jnp.float32),
        pltpu.VMEM((2, CHUNK, HH), jnp.float32),
        pltpu.SemaphoreType.DMA,
        pltpu.SemaphoreType.DMA,
        pltpu.VMEM_SHARED((N_PAD, HH), jnp.float32),
    ],
)
def _sc_spmm(yflat_hbm, eidx_hbm, ew_hbm, z2_hbm, idxb, ewb, gbuf,
             sem0, sem1, acc):
    c = lax.axis_index("c")
    s = lax.axis_index("s")
    off = (c * N_PAD).astype(jnp.int32)
    sems = (sem0, sem1)

    def stage(k, slot):
        pltpu.sync_copy(eidx_hbm.at[s, k], idxb.at[slot])
        pltpu.sync_copy(ew_hbm.at[s, k], ewb.at[slot])
        for j in range(CHUNK // 16):
            sl = pl.ds(j * 16, 16)
            idxb[slot, 0, sl] = idxb[slot, 0, sl] + off

    def gather_start(slot):
        pltpu.async_copy(yflat_hbm.at[idxb.at[slot, 0]], gbuf.at[slot],
                         sems[slot])

    def gather_wait(slot):
        pltpu.make_async_copy(yflat_hbm.at[idxb.at[slot, 0]], gbuf.at[slot],
                              sems[slot]).wait()

    def scale_scatter(slot):
        def inner(g, carry):
            wv = ewb[slot, pl.ds(g * 16, 16)]
            for l in range(16):
                e = g * 16 + l
                w = wv[l]
                for j in range(HH // 16):
                    sl = pl.ds(j * 16, 16)
                    gbuf[slot, e, sl] = gbuf[slot, e, sl] * w
            return carry

        lax.fori_loop(0, CHUNK // 16, inner, 0)
        pltpu.sync_copy(gbuf.at[slot], acc.at[idxb.at[slot, 1]], add=True)

    def zrow(r, carry):
        for j in range(HH // 16):
            gbuf[0, r, pl.ds(j * 16, 16)] = jnp.zeros((16,), jnp.float32)
        return carry

    lax.fori_loop(0, CHUNK, zrow, 0)
    for m in range(NB // CHUNK):
        pltpu.sync_copy(gbuf.at[0], acc.at[pl.ds(s * NB + m * CHUNK, CHUNK)])
    plsc.subcore_barrier()

    stage(0, 0)
    stage(1, 1)
    gather_start(0)

    def body(k, carry):
        even = (k % 2) == 0
        more = k + 1 < NCH
        more2 = k + 2 < NCH

        @pl.when(jnp.logical_and(even, more))
        def _():
            gather_start(1)

        @pl.when(jnp.logical_and(jnp.logical_not(even), more))
        def _():
            gather_start(0)

        @pl.when(even)
        def _():
            gather_wait(0)
            scale_scatter(0)

            @pl.when(more2)
            def _():
                stage(k + 2, 0)

        @pl.when(jnp.logical_not(even))
        def _():
            gather_wait(1)
            scale_scatter(1)

            @pl.when(more2)
            def _():
                stage(k + 2, 1)

        return carry

    lax.fori_loop(0, NCH, body, 0)
    plsc.subcore_barrier()
    pltpu.sync_copy(acc.at[pl.ds(s * NB, NB)], z2_hbm.at[c, pl.ds(s * NB, NB)])


def _full(shape):
    return pl.BlockSpec(shape, lambda i: tuple(0 for _ in shape))


def _tk_in_body(x_ref, win_ref, bin_ref, w0_ref, degp_ref,
                h_ref, dis_ref, y2_ref):
    h = jnp.dot(x_ref[...], win_ref[...], preferred_element_type=jnp.float32)
    h = h + bin_ref[...]
    dis = lax.rsqrt(1.0 + degp_ref[0] + degp_ref[1])
    h_ref[...] = h
    dis_ref[...] = dis
    y = dis * jnp.dot(h, w0_ref[...], preferred_element_type=jnp.float32)
    y2_ref[0] = y[:, :HH]
    y2_ref[1] = y[:, HH:]


def _tk_in(xp, W_in, b_in, W0, degp3):
    return pl.pallas_call(
        _tk_in_body,
        grid=(GRID,),
        in_specs=[
            pl.BlockSpec((NB, 128), lambda i: (i, 0)),
            _full((128, 256)),
            _full((1, 256)),
            _full((256, 256)),
            pl.BlockSpec((2, NB, 1), lambda i: (0, i, 0)),
        ],
        out_specs=[
            pl.BlockSpec((NB, 256), lambda i: (i, 0)),
            pl.BlockSpec((NB, 1), lambda i: (i, 0)),
            pl.BlockSpec((2, NB, HH), lambda i: (0, i, 0)),
        ],
        out_shape=[
            jax.ShapeDtypeStruct((N_PAD, 256), jnp.float32),
            jax.ShapeDtypeStruct((N_PAD, 1), jnp.float32),
            jax.ShapeDtypeStruct((2, N_PAD, HH), jnp.float32),
        ],
    )(xp, W_in, b_in, W0, degp3)


def _tk_stats_body(z2_ref, y2_ref, h_ref, dis_ref, b_ref, t_ref, st_ref):
    i = pl.program_id(0)
    z = jnp.concatenate([z2_ref[0], z2_ref[1]], axis=1)
    y = jnp.concatenate([y2_ref[0], y2_ref[1]], axis=1)
    t = dis_ref[...] * (z + y) + b_ref[...] + h_ref[...]
    t_ref[...] = t
    rows = i * NB + lax.broadcasted_iota(jnp.int32, (NB, 1), 0)
    tm = jnp.where(rows < N, t, 0.0)
    s1 = jnp.sum(tm, axis=0, keepdims=True)
    s2 = jnp.sum(tm * tm, axis=0, keepdims=True)
    blk = jnp.concatenate([s1, s2], axis=0)

    @pl.when(i == 0)
    def _():
        st_ref[...] = blk

    @pl.when(i > 0)
    def _():
        st_ref[...] = st_ref[...] + blk


def _tk_stats(z2, y2, h, dis, b):
    return pl.pallas_call(
        _tk_stats_body,
        grid=(GRID,),
        in_specs=[
            pl.BlockSpec((2, NB, HH), lambda i: (0, i, 0)),
            pl.BlockSpec((2, NB, HH), lambda i: (0, i, 0)),
            pl.BlockSpec((NB, 256), lambda i: (i, 0)),
            pl.BlockSpec((NB, 1), lambda i: (i, 0)),
            _full((1, 256)),
        ],
        out_specs=[
            pl.BlockSpec((NB, 256), lambda i: (i, 0)),
            _full((2, 256)),
        ],
        out_shape=[
            jax.ShapeDtypeStruct((N_PAD, 256), jnp.float32),
            jax.ShapeDtypeStruct((2, 256), jnp.float32),
        ],
    )(z2, y2, h, dis, b)


def _bn_relu(t_ref, st_ref, gam_ref, bet_ref):
    mu = st_ref[0:1, :] * (1.0 / N)
    var = st_ref[1:2, :] * (1.0 / N) - mu * mu
    inv = lax.rsqrt(var + 1e-5)
    return jnp.maximum(gam_ref[...] * (t_ref[...] - mu) * inv + bet_ref[...], 0.0)


def _tk_norm_body(t_ref, st_ref, gam_ref, bet_ref, dis_ref, w_ref,
                  h_ref, y2_ref):
    hn = _bn_relu(t_ref, st_ref, gam_ref, bet_ref)
    h_ref[...] = hn
    y = dis_ref[...] * jnp.dot(hn, w_ref[...], preferred_element_type=jnp.float32)
    y2_ref[0] = y[:, :HH]
    y2_ref[1] = y[:, HH:]


def _tk_norm(t, st, gam, bet, dis, Wn):
    return pl.pallas_call(
        _tk_norm_body,
        grid=(GRID,),
        in_specs=[
            pl.BlockSpec((NB, 256), lambda i: (i, 0)),
            _full((2, 256)),
            _full((1, 256)),
            _full((1, 256)),
            pl.BlockSpec((NB, 1), lambda i: (i, 0)),
            _full((256, 256)),
        ],
        out_specs=[
            pl.BlockSpec((NB, 256), lambda i: (i, 0)),
            pl.BlockSpec((2, NB, HH), lambda i: (0, i, 0)),
        ],
        out_shape=[
            jax.ShapeDtypeStruct((N_PAD, 256), jnp.float32),
            jax.ShapeDtypeStruct((2, N_PAD, HH), jnp.float32),
        ],
    )(t, st, gam, bet, dis, Wn)


def _tk_pool_body(t_ref, st_ref, gam_ref, bet_ref, bat_ref, ps_ref, cn_ref):
    i = pl.program_id(0)
    hn = _bn_relu(t_ref, st_ref, gam_ref, bet_ref)
    oh = (bat_ref[...] == lax.broadcasted_iota(jnp.int32, (NB, NGRP), 1))
    oh = oh.astype(jnp.float32)
    ps = lax.dot_general(oh, hn, (((0,), (0,)), ((), ())),
                         preferred_element_type=jnp.float32)
    cn = lax.dot_general(oh, jnp.ones((NB, 1), jnp.float32),
                         (((0,), (0,)), ((), ())),
                         preferred_element_type=jnp.float32)

    @pl.when(i == 0)
    def _():
        ps_ref[...] = ps
        cn_ref[...] = cn

    @pl.when(i > 0)
    def _():
        ps_ref[...] = ps_ref[...] + ps
        cn_ref[...] = cn_ref[...] + cn


def _tk_pool(t, st, gam, bet, batp):
    return pl.pallas_call(
        _tk_pool_body,
        grid=(GRID,),
        in_specs=[
            pl.BlockSpec((NB, 256), lambda i: (i, 0)),
            _full((2, 256)),
            _full((1, 256)),
            _full((1, 256)),
            pl.BlockSpec((NB, 1), lambda i: (i, 0)),
        ],
        out_specs=[
            _full((NGRP, 256)),
            _full((NGRP, 1)),
        ],
        out_shape=[
            jax.ShapeDtypeStruct((NGRP, 256), jnp.float32),
            jax.ShapeDtypeStruct((NGRP, 1), jnp.float32),
        ],
    )(t, st, gam, bet, batp)


def _tk_head_body(ps_ref, cn_ref, w1_ref, b1_ref, w2_ref, b2_ref,
                  w3_ref, b3_ref, o_ref):
    pooled = ps_ref[...] / jnp.maximum(cn_ref[...], 1.0)
    o = jnp.maximum(pooled, 0.0)
    o = jnp.dot(o, w1_ref[...], preferred_element_type=jnp.float32) + b1_ref[...]
    o = jnp.maximum(o, 0.0)
    o = jnp.dot(o, w2_ref[...], preferred_element_type=jnp.float32) + b2_ref[...]
    o = jnp.maximum(o, 0.0)
    o_ref[...] = jnp.dot(o, w3_ref[...], preferred_element_type=jnp.float32) + b3_ref[...]


def _tk_head(ps, cn, W1, b1, W2, b2, W3, b3):
    return pl.pallas_call(
        _tk_head_body,
        out_shape=jax.ShapeDtypeStruct((NGRP, 10), jnp.float32),
    )(ps, cn, W1, b1, W2, b2, W3, b3)


def kernel(x, edge_index, edge_attr, batch, W_in, b_in, W_gcn, b_gcn,
           bn_gamma, bn_beta, W1, b1, W2, b2, W3, b3):
    row = edge_index[0].astype(jnp.int32)
    col = edge_index[1].astype(jnp.int32)
    ew = edge_attr.astype(jnp.float32)

    npad = E_PAD - E
    spread = (jnp.arange(npad, dtype=jnp.int32) * 37) % N
    row_p = jnp.concatenate([row, spread])
    col_p = jnp.concatenate([col, spread])
    ew_p = jnp.concatenate([ew, jnp.zeros((npad,), jnp.float32)])

    col_d = col_p.reshape(32, NCH_DEG, CHUNK)
    ew_d = ew_p.reshape(32, NCH_DEG, CHUNK)
    eidx = jnp.stack([row_p.reshape(16, NCH, CHUNK),
                      col_p.reshape(16, NCH, CHUNK)], axis=2)
    ew_s = ew_p.reshape(16, NCH, CHUNK)

    degp = _sc_deg(col_d, ew_d)
    degp3 = degp.reshape(2, N_PAD, 1)

    xp = jnp.pad(x, ((0, N_PAD - N), (0, 0)))
    batp = jnp.pad(batch.astype(jnp.int32), (0, N_PAD - N),
                   constant_values=NGRP).reshape(N_PAD, 1)

    h, dis, y2 = _tk_in(xp, W_in, b_in.reshape(1, 256), W_gcn[0], degp3)

    ps = cn = None
    for i in range(4):
        z2 = _sc_spmm(y2.reshape(2 * N_PAD, HH), eidx, ew_s)
        t, st = _tk_stats(z2, y2, h, dis, b_gcn[i].reshape(1, 256))
        gam = bn_gamma[i].reshape(1, 256)
        bet = bn_beta[i].reshape(1, 256)
        if i < 3:
            h, y2 = _tk_norm(t, st, gam, bet, dis, W_gcn[i + 1])
        else:
            ps, cn = _tk_pool(t, st, gam, bet, batp)

    return _tk_head(ps, cn, W1, b1.reshape(1, 128), W2, b2.reshape(1, 64),
                    W3, b3.reshape(1, 10))


# speedup vs baseline: 13.4899x; 1.2416x over previous
import functools

import jax
import jax.numpy as jnp
from jax import lax
from jax.experimental import pallas as pl
from jax.experimental.pallas import tpu as pltpu
from jax.experimental.pallas import tpu_sc as plsc

N = 10000
N_PAD = 10240
NB = 640
GRID = N_PAD // NB
E = 320000
E_PAD = 323584
CHUNK = 128
NCH = E_PAD // 16 // CHUNK
NCH_DEG = E_PAD // 32 // CHUNK
HH = 128
NGRP = 64

_MESH = plsc.VectorSubcoreMesh(core_axis_name="c", subcore_axis_name="s")


@functools.partial(
    pl.kernel,
    out_type=jax.ShapeDtypeStruct((2, N_PAD), jnp.float32),
    mesh=_MESH,
    scratch_types=[
        pltpu.VMEM((NCH_DEG, CHUNK), jnp.int32),
        pltpu.VMEM((NCH_DEG, CHUNK), jnp.float32),
        pltpu.VMEM((NB,), jnp.float32),
        pltpu.VMEM_SHARED((N_PAD,), jnp.float32),
    ],
)
def _sc_deg(col_hbm, ew_hbm, degp_hbm, colv, ewv, zv, acc):
    c = lax.axis_index("c")
    s = lax.axis_index("s")
    wid = c * 16 + s
    for j in range(NB // 16):
        zv[pl.ds(j * 16, 16)] = jnp.zeros((16,), jnp.float32)
    pltpu.sync_copy(zv, acc.at[pl.ds(s * NB, NB)])
    plsc.subcore_barrier()
    pltpu.sync_copy(col_hbm.at[wid], colv)
    pltpu.sync_copy(ew_hbm.at[wid], ewv)

    def body(k, carry):
        pltpu.sync_copy(ewv.at[k], acc.at[colv.at[k]], add=True)
        return carry

    lax.fori_loop(0, NCH_DEG, body, 0)
    plsc.subcore_barrier()
    pltpu.sync_copy(acc.at[pl.ds(s * NB, NB)], degp_hbm.at[c, pl.ds(s * NB, NB)])


@functools.partial(
    pl.kernel,
    out_type=jax.ShapeDtypeStruct((2, N_PAD, HH), jnp.float32),
    mesh=_MESH,
    scratch_types=[
        pltpu.VMEM((2, 3, CHUNK), jnp.int32),
        pltpu.VMEM((2, CHUNK), jnp.float32),
        pltpu.VMEM((2, CHUNK), jnp.int32),
        pltpu.VMEM((2, CHUNK, HH), jnp.float32),
        pltpu.SemaphoreType.DMA,
        pltpu.SemaphoreType.DMA,
        pltpu.SemaphoreType.DMA,
        pltpu.SemaphoreType.DMA,
        pltpu.VMEM_SHARED((N_PAD, HH), jnp.float32),
    ],
)
def _sc_spmm(yflat_hbm, eidx_hbm, ew_hbm, z2_hbm, idxb, ewb, scol, gbuf,
             sem0, sem1, ssem0, ssem1, acc):
    c = lax.axis_index("c")
    s = lax.axis_index("s")
    sems = (sem0, sem1)
    ssems = (ssem0, ssem1)

    def stage(k, slot):
        pltpu.sync_copy(eidx_hbm.at[s, k], idxb.at[slot])
        pltpu.sync_copy(ew_hbm.at[s, k], ewb.at[slot])

    def gather_start(slot):
        pltpu.async_copy(yflat_hbm.at[idxb.at[slot, c]], gbuf.at[slot],
                         sems[slot])

    def gather_wait(slot):
        pltpu.make_async_copy(yflat_hbm.at[idxb.at[slot, c]], gbuf.at[slot],
                              sems[slot]).wait()

    def scale(slot):
        def inner(g, carry):
            wv = ewb[slot, pl.ds(g * 16, 16)]
            for l in range(16):
                e = g * 16 + l
                w = wv[l]
                for j in range(HH // 16):
                    sl = pl.ds(j * 16, 16)
                    gbuf[slot, e, sl] = gbuf[slot, e, sl] * w
            return carry

        lax.fori_loop(0, CHUNK // 16, inner, 0)

    def scatter_start(slot):
        for j in range(CHUNK // 16):
            sl = pl.ds(j * 16, 16)
            scol[slot, sl] = idxb[slot, 2, sl]
        pltpu.async_copy(gbuf.at[slot], acc.at[scol.at[slot]], ssems[slot],
                         add=True)

    def scatter_wait(slot):
        pltpu.make_async_copy(gbuf.at[slot], acc.at[scol.at[slot]],
                              ssems[slot]).wait()

    def zrow(r, carry):
        for j in range(HH // 16):
            gbuf[0, r, pl.ds(j * 16, 16)] = jnp.zeros((16,), jnp.float32)
        return carry

    lax.fori_loop(0, CHUNK, zrow, 0)
    for m in range(NB // CHUNK):
        pltpu.sync_copy(gbuf.at[0], acc.at[pl.ds(s * NB + m * CHUNK, CHUNK)])
    plsc.subcore_barrier()

    stage(0, 0)
    stage(1, 1)
    gather_start(0)

    def phase(k, slot, oslot):
        @pl.when(k >= 1)
        def _():
            scatter_wait(oslot)

        @pl.when(k + 1 < NCH)
        def _():
            gather_start(oslot)

        gather_wait(slot)
        scale(slot)
        scatter_start(slot)

        @pl.when(k + 2 < NCH)
        def _():
            stage(k + 2, slot)

    def body(k, carry):
        even = (k % 2) == 0

        @pl.when(even)
        def _():
            phase(k, 0, 1)

        @pl.when(jnp.logical_not(even))
        def _():
            phase(k, 1, 0)

        return carry

    lax.fori_loop(0, NCH, body, 0)
    scatter_wait((NCH - 1) % 2)
    plsc.subcore_barrier()
    pltpu.sync_copy(acc.at[pl.ds(s * NB, NB)], z2_hbm.at[c, pl.ds(s * NB, NB)])


def _full(shape):
    return pl.BlockSpec(shape, lambda i: tuple(0 for _ in shape))


def _tk_in_body(x_ref, win_ref, bin_ref, w0_ref, degp_ref,
                h_ref, dis_ref, y2_ref):
    h = jnp.dot(x_ref[...], win_ref[...], preferred_element_type=jnp.float32)
    h = h + bin_ref[...]
    dis = lax.rsqrt(1.0 + degp_ref[0] + degp_ref[1])
    h_ref[...] = h
    dis_ref[...] = dis
    y = dis * jnp.dot(h, w0_ref[...], preferred_element_type=jnp.float32)
    y2_ref[0] = y[:, :HH]
    y2_ref[1] = y[:, HH:]


def _tk_in(xp, W_in, b_in, W0, degp3):
    return pl.pallas_call(
        _tk_in_body,
        grid=(GRID,),
        in_specs=[
            pl.BlockSpec((NB, 128), lambda i: (i, 0)),
            _full((128, 256)),
            _full((1, 256)),
            _full((256, 256)),
            pl.BlockSpec((2, NB, 1), lambda i: (0, i, 0)),
        ],
        out_specs=[
            pl.BlockSpec((NB, 256), lambda i: (i, 0)),
            pl.BlockSpec((NB, 1), lambda i: (i, 0)),
            pl.BlockSpec((2, NB, HH), lambda i: (0, i, 0)),
        ],
        out_shape=[
            jax.ShapeDtypeStruct((N_PAD, 256), jnp.float32),
            jax.ShapeDtypeStruct((N_PAD, 1), jnp.float32),
            jax.ShapeDtypeStruct((2, N_PAD, HH), jnp.float32),
        ],
    )(xp, W_in, b_in, W0, degp3)


def _tk_stats_body(z2_ref, y2_ref, h_ref, dis_ref, b_ref, t_ref, st_ref):
    i = pl.program_id(0)
    z = jnp.concatenate([z2_ref[0], z2_ref[1]], axis=1)
    y = jnp.concatenate([y2_ref[0], y2_ref[1]], axis=1)
    t = dis_ref[...] * (z + y) + b_ref[...] + h_ref[...]
    t_ref[...] = t
    rows = i * NB + lax.broadcasted_iota(jnp.int32, (NB, 1), 0)
    tm = jnp.where(rows < N, t, 0.0)
    s1 = jnp.sum(tm, axis=0, keepdims=True)
    s2 = jnp.sum(tm * tm, axis=0, keepdims=True)
    blk = jnp.concatenate([s1, s2], axis=0)

    @pl.when(i == 0)
    def _():
        st_ref[...] = blk

    @pl.when(i > 0)
    def _():
        st_ref[...] = st_ref[...] + blk


def _tk_stats(z2, y2, h, dis, b):
    return pl.pallas_call(
        _tk_stats_body,
        grid=(GRID,),
        in_specs=[
            pl.BlockSpec((2, NB, HH), lambda i: (0, i, 0)),
            pl.BlockSpec((2, NB, HH), lambda i: (0, i, 0)),
            pl.BlockSpec((NB, 256), lambda i: (i, 0)),
            pl.BlockSpec((NB, 1), lambda i: (i, 0)),
            _full((1, 256)),
        ],
        out_specs=[
            pl.BlockSpec((NB, 256), lambda i: (i, 0)),
            _full((2, 256)),
        ],
        out_shape=[
            jax.ShapeDtypeStruct((N_PAD, 256), jnp.float32),
            jax.ShapeDtypeStruct((2, 256), jnp.float32),
        ],
    )(z2, y2, h, dis, b)


def _bn_relu(t_ref, st_ref, gam_ref, bet_ref):
    mu = st_ref[0:1, :] * (1.0 / N)
    var = st_ref[1:2, :] * (1.0 / N) - mu * mu
    inv = lax.rsqrt(var + 1e-5)
    return jnp.maximum(gam_ref[...] * (t_ref[...] - mu) * inv + bet_ref[...], 0.0)


def _tk_norm_body(t_ref, st_ref, gam_ref, bet_ref, dis_ref, w_ref,
                  h_ref, y2_ref):
    hn = _bn_relu(t_ref, st_ref, gam_ref, bet_ref)
    h_ref[...] = hn
    y = dis_ref[...] * jnp.dot(hn, w_ref[...], preferred_element_type=jnp.float32)
    y2_ref[0] = y[:, :HH]
    y2_ref[1] = y[:, HH:]


def _tk_norm(t, st, gam, bet, dis, Wn):
    return pl.pallas_call(
        _tk_norm_body,
        grid=(GRID,),
        in_specs=[
            pl.BlockSpec((NB, 256), lambda i: (i, 0)),
            _full((2, 256)),
            _full((1, 256)),
            _full((1, 256)),
            pl.BlockSpec((NB, 1), lambda i: (i, 0)),
            _full((256, 256)),
        ],
        out_specs=[
            pl.BlockSpec((NB, 256), lambda i: (i, 0)),
            pl.BlockSpec((2, NB, HH), lambda i: (0, i, 0)),
        ],
        out_shape=[
            jax.ShapeDtypeStruct((N_PAD, 256), jnp.float32),
            jax.ShapeDtypeStruct((2, N_PAD, HH), jnp.float32),
        ],
    )(t, st, gam, bet, dis, Wn)


def _tk_pool_body(t_ref, st_ref, gam_ref, bet_ref, bat_ref, ps_ref, cn_ref):
    i = pl.program_id(0)
    hn = _bn_relu(t_ref, st_ref, gam_ref, bet_ref)
    oh = (bat_ref[...] == lax.broadcasted_iota(jnp.int32, (NB, NGRP), 1))
    oh = oh.astype(jnp.float32)
    ps = lax.dot_general(oh, hn, (((0,), (0,)), ((), ())),
                         preferred_element_type=jnp.float32)
    cn = lax.dot_general(oh, jnp.ones((NB, 1), jnp.float32),
                         (((0,), (0,)), ((), ())),
                         preferred_element_type=jnp.float32)

    @pl.when(i == 0)
    def _():
        ps_ref[...] = ps
        cn_ref[...] = cn

    @pl.when(i > 0)
    def _():
        ps_ref[...] = ps_ref[...] + ps
        cn_ref[...] = cn_ref[...] + cn


def _tk_pool(t, st, gam, bet, batp):
    return pl.pallas_call(
        _tk_pool_body,
        grid=(GRID,),
        in_specs=[
            pl.BlockSpec((NB, 256), lambda i: (i, 0)),
            _full((2, 256)),
            _full((1, 256)),
            _full((1, 256)),
            pl.BlockSpec((NB, 1), lambda i: (i, 0)),
        ],
        out_specs=[
            _full((NGRP, 256)),
            _full((NGRP, 1)),
        ],
        out_shape=[
            jax.ShapeDtypeStruct((NGRP, 256), jnp.float32),
            jax.ShapeDtypeStruct((NGRP, 1), jnp.float32),
        ],
    )(t, st, gam, bet, batp)


def _tk_head_body(ps_ref, cn_ref, w1_ref, b1_ref, w2_ref, b2_ref,
                  w3_ref, b3_ref, o_ref):
    pooled = ps_ref[...] / jnp.maximum(cn_ref[...], 1.0)
    o = jnp.maximum(pooled, 0.0)
    o = jnp.dot(o, w1_ref[...], preferred_element_type=jnp.float32) + b1_ref[...]
    o = jnp.maximum(o, 0.0)
    o = jnp.dot(o, w2_ref[...], preferred_element_type=jnp.float32) + b2_ref[...]
    o = jnp.maximum(o, 0.0)
    o_ref[...] = jnp.dot(o, w3_ref[...], preferred_element_type=jnp.float32) + b3_ref[...]


def _tk_head(ps, cn, W1, b1, W2, b2, W3, b3):
    return pl.pallas_call(
        _tk_head_body,
        out_shape=jax.ShapeDtypeStruct((NGRP, 10), jnp.float32),
    )(ps, cn, W1, b1, W2, b2, W3, b3)


def kernel(x, edge_index, edge_attr, batch, W_in, b_in, W_gcn, b_gcn,
           bn_gamma, bn_beta, W1, b1, W2, b2, W3, b3):
    row = edge_index[0].astype(jnp.int32)
    col = edge_index[1].astype(jnp.int32)
    ew = edge_attr.astype(jnp.float32)

    npad = E_PAD - E
    spread = (jnp.arange(npad, dtype=jnp.int32) * 37) % N
    row_p = jnp.concatenate([row, spread])
    col_p = jnp.concatenate([col, spread])
    ew_p = jnp.concatenate([ew, jnp.zeros((npad,), jnp.float32)])

    col_d = col_p.reshape(32, NCH_DEG, CHUNK)
    ew_d = ew_p.reshape(32, NCH_DEG, CHUNK)
    eidx = jnp.stack([row_p.reshape(16, NCH, CHUNK),
                      (row_p + N_PAD).reshape(16, NCH, CHUNK),
                      col_p.reshape(16, NCH, CHUNK)], axis=2)
    ew_s = ew_p.reshape(16, NCH, CHUNK)

    degp = _sc_deg(col_d, ew_d)
    degp3 = degp.reshape(2, N_PAD, 1)

    xp = jnp.pad(x, ((0, N_PAD - N), (0, 0)))
    batp = jnp.pad(batch.astype(jnp.int32), (0, N_PAD - N),
                   constant_values=NGRP).reshape(N_PAD, 1)

    h, dis, y2 = _tk_in(xp, W_in, b_in.reshape(1, 256), W_gcn[0], degp3)

    ps = cn = None
    for i in range(4):
        z2 = _sc_spmm(y2.reshape(2 * N_PAD, HH), eidx, ew_s)
        t, st = _tk_stats(z2, y2, h, dis, b_gcn[i].reshape(1, 256))
        gam = bn_gamma[i].reshape(1, 256)
        bet = bn_beta[i].reshape(1, 256)
        if i < 3:
            h, y2 = _tk_norm(t, st, gam, bet, dis, W_gcn[i + 1])
        else:
            ps, cn = _tk_pool(t, st, gam, bet, batp)

    return _tk_head(ps, cn, W1, b1.reshape(1, 128), W2, b2.reshape(1, 64),
                    W3, b3.reshape(1, 10))


# speedup vs baseline: 15.6923x; 1.1633x over previous
import functools

import jax
import jax.numpy as jnp
from jax import lax
from jax.experimental import pallas as pl
from jax.experimental.pallas import tpu as pltpu
from jax.experimental.pallas import tpu_sc as plsc

N = 10000
N_PAD = 10240
NB = 640
GRID = N_PAD // NB
E = 320000
E_PAD = 323584
CHUNK = 128
NCH = E_PAD // 16 // CHUNK
NCH_DEG = E_PAD // 32 // CHUNK
HH = 128
NGRP = 64

_MESH = plsc.VectorSubcoreMesh(core_axis_name="c", subcore_axis_name="s")


@functools.partial(
    pl.kernel,
    out_type=jax.ShapeDtypeStruct((2, N_PAD), jnp.float32),
    mesh=_MESH,
    scratch_types=[
        pltpu.VMEM((NCH_DEG, CHUNK), jnp.int32),
        pltpu.VMEM((NCH_DEG, CHUNK), jnp.float32),
        pltpu.VMEM((NB,), jnp.float32),
        pltpu.VMEM_SHARED((N_PAD,), jnp.float32),
    ],
)
def _sc_deg(col_hbm, ew_hbm, degp_hbm, colv, ewv, zv, acc):
    c = lax.axis_index("c")
    s = lax.axis_index("s")
    wid = c * 16 + s
    for j in range(NB // 16):
        zv[pl.ds(j * 16, 16)] = jnp.zeros((16,), jnp.float32)
    pltpu.sync_copy(zv, acc.at[pl.ds(s * NB, NB)])
    plsc.subcore_barrier()
    pltpu.sync_copy(col_hbm.at[wid], colv)
    pltpu.sync_copy(ew_hbm.at[wid], ewv)

    def body(k, carry):
        pltpu.sync_copy(ewv.at[k], acc.at[colv.at[k]], add=True)
        return carry

    lax.fori_loop(0, NCH_DEG, body, 0)
    plsc.subcore_barrier()
    pltpu.sync_copy(acc.at[pl.ds(s * NB, NB)], degp_hbm.at[c, pl.ds(s * NB, NB)])


@functools.partial(
    pl.kernel,
    out_type=jax.ShapeDtypeStruct((2, N_PAD, HH), jnp.float32),
    mesh=_MESH,
    scratch_types=[
        pltpu.VMEM((4, 3, CHUNK), jnp.int32),
        pltpu.VMEM((4, CHUNK), jnp.float32),
        pltpu.VMEM((2, CHUNK), jnp.int32),
        pltpu.VMEM((2, CHUNK, HH), jnp.float32),
        pltpu.SemaphoreType.DMA,
        pltpu.SemaphoreType.DMA,
        pltpu.SemaphoreType.DMA,
        pltpu.SemaphoreType.DMA,
        pltpu.SemaphoreType.DMA,
        pltpu.VMEM_SHARED((N_PAD, HH), jnp.float32),
    ],
)
def _sc_spmm(yflat_hbm, eidx_hbm, ew_hbm, z2_hbm, idxb, ewb, scol, gbuf,
             sem0, sem1, ssem0, ssem1, stsem, acc):
    c = lax.axis_index("c")
    s = lax.axis_index("s")
    sems = (sem0, sem1)
    ssems = (ssem0, ssem1)

    def stage_start(k):
        r = k % 4
        pltpu.async_copy(eidx_hbm.at[s, k], idxb.at[r], stsem)
        pltpu.async_copy(ew_hbm.at[s, k], ewb.at[r], stsem)

    def stage_wait(k):
        r = k % 4
        pltpu.make_async_copy(eidx_hbm.at[s, k], idxb.at[r], stsem).wait()
        pltpu.make_async_copy(ew_hbm.at[s, k], ewb.at[r], stsem).wait()

    def gather_start(k, slot):
        pltpu.async_copy(yflat_hbm.at[idxb.at[k % 4, c]], gbuf.at[slot],
                         sems[slot])

    def gather_wait(k, slot):
        pltpu.make_async_copy(yflat_hbm.at[idxb.at[k % 4, c]], gbuf.at[slot],
                              sems[slot]).wait()

    def scale(k, slot):
        r = k % 4

        def inner(g, carry):
            wv = ewb[r, pl.ds(g * 16, 16)]
            for l in range(16):
                e = g * 16 + l
                w = wv[l]
                for j in range(HH // 16):
                    sl = pl.ds(j * 16, 16)
                    gbuf[slot, e, sl] = gbuf[slot, e, sl] * w
            return carry

        lax.fori_loop(0, CHUNK // 16, inner, 0)

    def scatter_start(k, slot):
        r = k % 4
        for j in range(CHUNK // 16):
            sl = pl.ds(j * 16, 16)
            scol[slot, sl] = idxb[r, 2, sl]
        pltpu.async_copy(gbuf.at[slot], acc.at[scol.at[slot]], ssems[slot],
                         add=True)

    def scatter_wait(slot):
        pltpu.make_async_copy(gbuf.at[slot], acc.at[scol.at[slot]],
                              ssems[slot]).wait()

    def zrow(r, carry):
        for j in range(HH // 16):
            gbuf[0, r, pl.ds(j * 16, 16)] = jnp.zeros((16,), jnp.float32)
        return carry

    lax.fori_loop(0, CHUNK, zrow, 0)
    for m in range(NB // CHUNK):
        pltpu.sync_copy(gbuf.at[0], acc.at[pl.ds(s * NB + m * CHUNK, CHUNK)])
    plsc.subcore_barrier()

    stage_start(0)
    stage_start(1)
    stage_start(2)
    stage_wait(0)
    gather_start(0, 0)

    def phase(k, slot, oslot):
        @pl.when(k >= 1)
        def _():
            scatter_wait(oslot)

        @pl.when(k + 1 < NCH)
        def _():
            stage_wait(k + 1)
            gather_start(k + 1, oslot)

        gather_wait(k, slot)
        scale(k, slot)
        scatter_start(k, slot)

        @pl.when(k + 3 < NCH)
        def _():
            stage_start(k + 3)

    def body(k, carry):
        even = (k % 2) == 0

        @pl.when(even)
        def _():
            phase(k, 0, 1)

        @pl.when(jnp.logical_not(even))
        def _():
            phase(k, 1, 0)

        return carry

    lax.fori_loop(0, NCH, body, 0)
    scatter_wait((NCH - 1) % 2)
    plsc.subcore_barrier()
    pltpu.sync_copy(acc.at[pl.ds(s * NB, NB)], z2_hbm.at[c, pl.ds(s * NB, NB)])


def _full(shape):
    return pl.BlockSpec(shape, lambda i: tuple(0 for _ in shape))


def _tk_in_body(x_ref, win_ref, bin_ref, w0_ref, degp_ref,
                h_ref, dis_ref, y2_ref):
    h = jnp.dot(x_ref[...], win_ref[...], preferred_element_type=jnp.float32)
    h = h + bin_ref[...]
    dis = lax.rsqrt(1.0 + degp_ref[0] + degp_ref[1])
    h_ref[...] = h
    dis_ref[...] = dis
    y = dis * jnp.dot(h, w0_ref[...], preferred_element_type=jnp.float32)
    y2_ref[0] = y[:, :HH]
    y2_ref[1] = y[:, HH:]


def _tk_in(xp, W_in, b_in, W0, degp3):
    return pl.pallas_call(
        _tk_in_body,
        grid=(GRID,),
        in_specs=[
            pl.BlockSpec((NB, 128), lambda i: (i, 0)),
            _full((128, 256)),
            _full((1, 256)),
            _full((256, 256)),
            pl.BlockSpec((2, NB, 1), lambda i: (0, i, 0)),
        ],
        out_specs=[
            pl.BlockSpec((NB, 256), lambda i: (i, 0)),
            pl.BlockSpec((NB, 1), lambda i: (i, 0)),
            pl.BlockSpec((2, NB, HH), lambda i: (0, i, 0)),
        ],
        out_shape=[
            jax.ShapeDtypeStruct((N_PAD, 256), jnp.float32),
            jax.ShapeDtypeStruct((N_PAD, 1), jnp.float32),
            jax.ShapeDtypeStruct((2, N_PAD, HH), jnp.float32),
        ],
    )(xp, W_in, b_in, W0, degp3)


def _tk_stats_body(z2_ref, y2_ref, h_ref, dis_ref, b_ref, t_ref, st_ref):
    i = pl.program_id(0)
    z = jnp.concatenate([z2_ref[0], z2_ref[1]], axis=1)
    y = jnp.concatenate([y2_ref[0], y2_ref[1]], axis=1)
    t = dis_ref[...] * (z + y) + b_ref[...] + h_ref[...]
    t_ref[...] = t
    rows = i * NB + lax.broadcasted_iota(jnp.int32, (NB, 1), 0)
    tm = jnp.where(rows < N, t, 0.0)
    s1 = jnp.sum(tm, axis=0, keepdims=True)
    s2 = jnp.sum(tm * tm, axis=0, keepdims=True)
    blk = jnp.concatenate([s1, s2], axis=0)

    @pl.when(i == 0)
    def _():
        st_ref[...] = blk

    @pl.when(i > 0)
    def _():
        st_ref[...] = st_ref[...] + blk


def _tk_stats(z2, y2, h, dis, b):
    return pl.pallas_call(
        _tk_stats_body,
        grid=(GRID,),
        in_specs=[
            pl.BlockSpec((2, NB, HH), lambda i: (0, i, 0)),
            pl.BlockSpec((2, NB, HH), lambda i: (0, i, 0)),
            pl.BlockSpec((NB, 256), lambda i: (i, 0)),
            pl.BlockSpec((NB, 1), lambda i: (i, 0)),
            _full((1, 256)),
        ],
        out_specs=[
            pl.BlockSpec((NB, 256), lambda i: (i, 0)),
            _full((2, 256)),
        ],
        out_shape=[
            jax.ShapeDtypeStruct((N_PAD, 256), jnp.float32),
            jax.ShapeDtypeStruct((2, 256), jnp.float32),
        ],
    )(z2, y2, h, dis, b)


def _bn_relu(t_ref, st_ref, gam_ref, bet_ref):
    mu = st_ref[0:1, :] * (1.0 / N)
    var = st_ref[1:2, :] * (1.0 / N) - mu * mu
    inv = lax.rsqrt(var + 1e-5)
    return jnp.maximum(gam_ref[...] * (t_ref[...] - mu) * inv + bet_ref[...], 0.0)


def _tk_norm_body(t_ref, st_ref, gam_ref, bet_ref, dis_ref, w_ref,
                  h_ref, y2_ref):
    hn = _bn_relu(t_ref, st_ref, gam_ref, bet_ref)
    h_ref[...] = hn
    y = dis_ref[...] * jnp.dot(hn, w_ref[...], preferred_element_type=jnp.float32)
    y2_ref[0] = y[:, :HH]
    y2_ref[1] = y[:, HH:]


def _tk_norm(t, st, gam, bet, dis, Wn):
    return pl.pallas_call(
        _tk_norm_body,
        grid=(GRID,),
        in_specs=[
            pl.BlockSpec((NB, 256), lambda i: (i, 0)),
            _full((2, 256)),
            _full((1, 256)),
            _full((1, 256)),
            pl.BlockSpec((NB, 1), lambda i: (i, 0)),
            _full((256, 256)),
        ],
        out_specs=[
            pl.BlockSpec((NB, 256), lambda i: (i, 0)),
            pl.BlockSpec((2, NB, HH), lambda i: (0, i, 0)),
        ],
        out_shape=[
            jax.ShapeDtypeStruct((N_PAD, 256), jnp.float32),
            jax.ShapeDtypeStruct((2, N_PAD, HH), jnp.float32),
        ],
    )(t, st, gam, bet, dis, Wn)


def _tk_pool_body(t_ref, st_ref, gam_ref, bet_ref, bat_ref, ps_ref, cn_ref):
    i = pl.program_id(0)
    hn = _bn_relu(t_ref, st_ref, gam_ref, bet_ref)
    oh = (bat_ref[...] == lax.broadcasted_iota(jnp.int32, (NB, NGRP), 1))
    oh = oh.astype(jnp.float32)
    ps = lax.dot_general(oh, hn, (((0,), (0,)), ((), ())),
                         preferred_element_type=jnp.float32)
    cn = lax.dot_general(oh, jnp.ones((NB, 1), jnp.float32),
                         (((0,), (0,)), ((), ())),
                         preferred_element_type=jnp.float32)

    @pl.when(i == 0)
    def _():
        ps_ref[...] = ps
        cn_ref[...] = cn

    @pl.when(i > 0)
    def _():
        ps_ref[...] = ps_ref[...] + ps
        cn_ref[...] = cn_ref[...] + cn


def _tk_pool(t, st, gam, bet, batp):
    return pl.pallas_call(
        _tk_pool_body,
        grid=(GRID,),
        in_specs=[
            pl.BlockSpec((NB, 256), lambda i: (i, 0)),
            _full((2, 256)),
            _full((1, 256)),
            _full((1, 256)),
            pl.BlockSpec((NB, 1), lambda i: (i, 0)),
        ],
        out_specs=[
            _full((NGRP, 256)),
            _full((NGRP, 1)),
        ],
        out_shape=[
            jax.ShapeDtypeStruct((NGRP, 256), jnp.float32),
            jax.ShapeDtypeStruct((NGRP, 1), jnp.float32),
        ],
    )(t, st, gam, bet, batp)


def _tk_head_body(ps_ref, cn_ref, w1_ref, b1_ref, w2_ref, b2_ref,
                  w3_ref, b3_ref, o_ref):
    pooled = ps_ref[...] / jnp.maximum(cn_ref[...], 1.0)
    o = jnp.maximum(pooled, 0.0)
    o = jnp.dot(o, w1_ref[...], preferred_element_type=jnp.float32) + b1_ref[...]
    o = jnp.maximum(o, 0.0)
    o = jnp.dot(o, w2_ref[...], preferred_element_type=jnp.float32) + b2_ref[...]
    o = jnp.maximum(o, 0.0)
    o_ref[...] = jnp.dot(o, w3_ref[...], preferred_element_type=jnp.float32) + b3_ref[...]


def _tk_head(ps, cn, W1, b1, W2, b2, W3, b3):
    return pl.pallas_call(
        _tk_head_body,
        out_shape=jax.ShapeDtypeStruct((NGRP, 10), jnp.float32),
    )(ps, cn, W1, b1, W2, b2, W3, b3)


def kernel(x, edge_index, edge_attr, batch, W_in, b_in, W_gcn, b_gcn,
           bn_gamma, bn_beta, W1, b1, W2, b2, W3, b3):
    row = edge_index[0].astype(jnp.int32)
    col = edge_index[1].astype(jnp.int32)
    ew = edge_attr.astype(jnp.float32)

    npad = E_PAD - E
    spread = (jnp.arange(npad, dtype=jnp.int32) * 37) % N
    row_p = jnp.concatenate([row, spread])
    col_p = jnp.concatenate([col, spread])
    ew_p = jnp.concatenate([ew, jnp.zeros((npad,), jnp.float32)])

    col_d = col_p.reshape(32, NCH_DEG, CHUNK)
    ew_d = ew_p.reshape(32, NCH_DEG, CHUNK)
    eidx = jnp.stack([row_p.reshape(16, NCH, CHUNK),
                      (row_p + N_PAD).reshape(16, NCH, CHUNK),
                      col_p.reshape(16, NCH, CHUNK)], axis=2)
    ew_s = ew_p.reshape(16, NCH, CHUNK)

    degp = _sc_deg(col_d, ew_d)
    degp3 = degp.reshape(2, N_PAD, 1)

    xp = jnp.pad(x, ((0, N_PAD - N), (0, 0)))
    batp = jnp.pad(batch.astype(jnp.int32), (0, N_PAD - N),
                   constant_values=NGRP).reshape(N_PAD, 1)

    h, dis, y2 = _tk_in(xp, W_in, b_in.reshape(1, 256), W_gcn[0], degp3)

    ps = cn = None
    for i in range(4):
        z2 = _sc_spmm(y2.reshape(2 * N_PAD, HH), eidx, ew_s)
        t, st = _tk_stats(z2, y2, h, dis, b_gcn[i].reshape(1, 256))
        gam = bn_gamma[i].reshape(1, 256)
        bet = bn_beta[i].reshape(1, 256)
        if i < 3:
            h, y2 = _tk_norm(t, st, gam, bet, dis, W_gcn[i + 1])
        else:
            ps, cn = _tk_pool(t, st, gam, bet, batp)

    return _tk_head(ps, cn, W1, b1.reshape(1, 128), W2, b2.reshape(1, 64),
                    W3, b3.reshape(1, 10))


# speedup vs baseline: 15.7582x; 1.0042x over previous
import functools

import jax
import jax.numpy as jnp
from jax import lax
from jax.experimental import pallas as pl
from jax.experimental.pallas import tpu as pltpu
from jax.experimental.pallas import tpu_sc as plsc

N = 10000
N_PAD = 10240
NB = 640
GRID = N_PAD // NB
E = 320000
E_PAD = 323584
CHUNK = 128
NCH = E_PAD // 16 // CHUNK
NCH_DEG = E_PAD // 32 // CHUNK
HH = 128
NGRP = 64

_MESH = plsc.VectorSubcoreMesh(core_axis_name="c", subcore_axis_name="s")


@functools.partial(
    pl.kernel,
    out_type=jax.ShapeDtypeStruct((2, N_PAD), jnp.float32),
    mesh=_MESH,
    scratch_types=[
        pltpu.VMEM((NCH_DEG, CHUNK), jnp.int32),
        pltpu.VMEM((NCH_DEG, CHUNK), jnp.float32),
        pltpu.VMEM((NB,), jnp.float32),
        pltpu.VMEM_SHARED((N_PAD,), jnp.float32),
    ],
)
def _sc_deg(col_hbm, ew_hbm, degp_hbm, colv, ewv, zv, acc):
    c = lax.axis_index("c")
    s = lax.axis_index("s")
    wid = c * 16 + s
    for j in range(NB // 16):
        zv[pl.ds(j * 16, 16)] = jnp.zeros((16,), jnp.float32)
    pltpu.sync_copy(zv, acc.at[pl.ds(s * NB, NB)])
    plsc.subcore_barrier()
    pltpu.sync_copy(col_hbm.at[wid], colv)
    pltpu.sync_copy(ew_hbm.at[wid], ewv)

    def body(k, carry):
        pltpu.sync_copy(ewv.at[k], acc.at[colv.at[k]], add=True)
        return carry

    lax.fori_loop(0, NCH_DEG, body, 0)
    plsc.subcore_barrier()
    pltpu.sync_copy(acc.at[pl.ds(s * NB, NB)], degp_hbm.at[c, pl.ds(s * NB, NB)])


@functools.partial(
    pl.kernel,
    out_type=jax.ShapeDtypeStruct((2, N_PAD, HH), jnp.float32),
    mesh=_MESH,
    scratch_types=[
        pltpu.VMEM((4, 3, CHUNK), jnp.int32),
        pltpu.VMEM((4, CHUNK), jnp.float32),
        pltpu.VMEM((2, CHUNK, HH), jnp.float32),
        pltpu.SemaphoreType.DMA,
        pltpu.SemaphoreType.DMA,
        pltpu.SemaphoreType.DMA,
        pltpu.SemaphoreType.DMA,
        pltpu.SemaphoreType.DMA,
        pltpu.VMEM_SHARED((N_PAD, HH), jnp.float32),
    ],
)
def _sc_spmm(yflat_hbm, eidx_hbm, ew_hbm, z2_hbm, idxb, ewb, gbuf,
             sem0, sem1, ssem0, ssem1, stsem, acc):
    c = lax.axis_index("c")
    s = lax.axis_index("s")
    sems = (sem0, sem1)
    ssems = (ssem0, ssem1)

    def stage_start(k):
        r = k % 4
        pltpu.async_copy(eidx_hbm.at[s, k], idxb.at[r], stsem)
        pltpu.async_copy(ew_hbm.at[s, k], ewb.at[r], stsem)

    def stage_wait(k):
        r = k % 4
        pltpu.make_async_copy(eidx_hbm.at[s, k], idxb.at[r], stsem).wait()
        pltpu.make_async_copy(ew_hbm.at[s, k], ewb.at[r], stsem).wait()

    def gather_start(k, slot):
        pltpu.async_copy(yflat_hbm.at[idxb.at[k % 4, c]], gbuf.at[slot],
                         sems[slot])

    def gather_wait(k, slot):
        pltpu.make_async_copy(yflat_hbm.at[idxb.at[k % 4, c]], gbuf.at[slot],
                              sems[slot]).wait()

    def scale(k, slot):
        r = k % 4

        def inner(g, carry):
            wv = ewb[r, pl.ds(g * 16, 16)]
            for l in range(16):
                e = g * 16 + l
                w = wv[l]
                for j in range(HH // 16):
                    sl = pl.ds(j * 16, 16)
                    gbuf[slot, e, sl] = gbuf[slot, e, sl] * w
            return carry

        lax.fori_loop(0, CHUNK // 16, inner, 0)

    def scatter_start(k, slot):
        pltpu.async_copy(gbuf.at[slot], acc.at[idxb.at[k % 4, 2]],
                         ssems[slot], add=True)

    def scatter_wait(k, slot):
        pltpu.make_async_copy(gbuf.at[slot], acc.at[idxb.at[k % 4, 2]],
                              ssems[slot]).wait()

    def zrow(r, carry):
        for j in range(HH // 16):
            gbuf[0, r, pl.ds(j * 16, 16)] = jnp.zeros((16,), jnp.float32)
        return carry

    lax.fori_loop(0, CHUNK, zrow, 0)
    for m in range(NB // CHUNK):
        pltpu.sync_copy(gbuf.at[0], acc.at[pl.ds(s * NB + m * CHUNK, CHUNK)])
    plsc.subcore_barrier()

    stage_start(0)
    stage_start(1)
    stage_start(2)
    stage_wait(0)
    gather_start(0, 0)

    def phase(k, slot, oslot):
        @pl.when(k >= 1)
        def _():
            scatter_wait(k - 1, oslot)

        @pl.when(k + 1 < NCH)
        def _():
            stage_wait(k + 1)
            gather_start(k + 1, oslot)

        gather_wait(k, slot)
        scale(k, slot)
        scatter_start(k, slot)

        @pl.when(k + 3 < NCH)
        def _():
            stage_start(k + 3)

    def body(k, carry):
        even = (k % 2) == 0

        @pl.when(even)
        def _():
            phase(k, 0, 1)

        @pl.when(jnp.logical_not(even))
        def _():
            phase(k, 1, 0)

        return carry

    lax.fori_loop(0, NCH, body, 0)
    scatter_wait(NCH - 1, (NCH - 1) % 2)
    plsc.subcore_barrier()
    pltpu.sync_copy(acc.at[pl.ds(s * NB, NB)], z2_hbm.at[c, pl.ds(s * NB, NB)])


def _full(shape):
    return pl.BlockSpec(shape, lambda i: tuple(0 for _ in shape))


def _tk_in_body(x_ref, win_ref, bin_ref, w0_ref, degp_ref,
                h_ref, dis_ref, y2_ref):
    h = jnp.dot(x_ref[...], win_ref[...], preferred_element_type=jnp.float32)
    h = h + bin_ref[...]
    dis = lax.rsqrt(1.0 + degp_ref[0] + degp_ref[1])
    h_ref[...] = h
    dis_ref[...] = dis
    y = dis * jnp.dot(h, w0_ref[...], preferred_element_type=jnp.float32)
    y2_ref[0] = y[:, :HH]
    y2_ref[1] = y[:, HH:]


def _tk_in(xp, W_in, b_in, W0, degp3):
    return pl.pallas_call(
        _tk_in_body,
        grid=(GRID,),
        in_specs=[
            pl.BlockSpec((NB, 128), lambda i: (i, 0)),
            _full((128, 256)),
            _full((1, 256)),
            _full((256, 256)),
            pl.BlockSpec((2, NB, 1), lambda i: (0, i, 0)),
        ],
        out_specs=[
            pl.BlockSpec((NB, 256), lambda i: (i, 0)),
            pl.BlockSpec((NB, 1), lambda i: (i, 0)),
            pl.BlockSpec((2, NB, HH), lambda i: (0, i, 0)),
        ],
        out_shape=[
            jax.ShapeDtypeStruct((N_PAD, 256), jnp.float32),
            jax.ShapeDtypeStruct((N_PAD, 1), jnp.float32),
            jax.ShapeDtypeStruct((2, N_PAD, HH), jnp.float32),
        ],
    )(xp, W_in, b_in, W0, degp3)


def _tk_stats_body(z2_ref, y2_ref, h_ref, dis_ref, b_ref, t_ref, st_ref):
    i = pl.program_id(0)
    z = jnp.concatenate([z2_ref[0], z2_ref[1]], axis=1)
    y = jnp.concatenate([y2_ref[0], y2_ref[1]], axis=1)
    t = dis_ref[...] * (z + y) + b_ref[...] + h_ref[...]
    t_ref[...] = t
    rows = i * NB + lax.broadcasted_iota(jnp.int32, (NB, 1), 0)
    tm = jnp.where(rows < N, t, 0.0)
    s1 = jnp.sum(tm, axis=0, keepdims=True)
    s2 = jnp.sum(tm * tm, axis=0, keepdims=True)
    blk = jnp.concatenate([s1, s2], axis=0)

    @pl.when(i == 0)
    def _():
        st_ref[...] = blk

    @pl.when(i > 0)
    def _():
        st_ref[...] = st_ref[...] + blk


def _tk_stats(z2, y2, h, dis, b):
    return pl.pallas_call(
        _tk_stats_body,
        grid=(GRID,),
        in_specs=[
            pl.BlockSpec((2, NB, HH), lambda i: (0, i, 0)),
            pl.BlockSpec((2, NB, HH), lambda i: (0, i, 0)),
            pl.BlockSpec((NB, 256), lambda i: (i, 0)),
            pl.BlockSpec((NB, 1), lambda i: (i, 0)),
            _full((1, 256)),
        ],
        out_specs=[
            pl.BlockSpec((NB, 256), lambda i: (i, 0)),
            _full((2, 256)),
        ],
        out_shape=[
            jax.ShapeDtypeStruct((N_PAD, 256), jnp.float32),
            jax.ShapeDtypeStruct((2, 256), jnp.float32),
        ],
    )(z2, y2, h, dis, b)


def _bn_relu(t_ref, st_ref, gam_ref, bet_ref):
    mu = st_ref[0:1, :] * (1.0 / N)
    var = st_ref[1:2, :] * (1.0 / N) - mu * mu
    inv = lax.rsqrt(var + 1e-5)
    return jnp.maximum(gam_ref[...] * (t_ref[...] - mu) * inv + bet_ref[...], 0.0)


def _tk_norm_body(t_ref, st_ref, gam_ref, bet_ref, dis_ref, w_ref,
                  h_ref, y2_ref):
    hn = _bn_relu(t_ref, st_ref, gam_ref, bet_ref)
    h_ref[...] = hn
    y = dis_ref[...] * jnp.dot(hn, w_ref[...], preferred_element_type=jnp.float32)
    y2_ref[0] = y[:, :HH]
    y2_ref[1] = y[:, HH:]


def _tk_norm(t, st, gam, bet, dis, Wn):
    return pl.pallas_call(
        _tk_norm_body,
        grid=(GRID,),
        in_specs=[
            pl.BlockSpec((NB, 256), lambda i: (i, 0)),
            _full((2, 256)),
            _full((1, 256)),
            _full((1, 256)),
            pl.BlockSpec((NB, 1), lambda i: (i, 0)),
            _full((256, 256)),
        ],
        out_specs=[
            pl.BlockSpec((NB, 256), lambda i: (i, 0)),
            pl.BlockSpec((2, NB, HH), lambda i: (0, i, 0)),
        ],
        out_shape=[
            jax.ShapeDtypeStruct((N_PAD, 256), jnp.float32),
            jax.ShapeDtypeStruct((2, N_PAD, HH), jnp.float32),
        ],
    )(t, st, gam, bet, dis, Wn)


def _tk_pool_body(t_ref, st_ref, gam_ref, bet_ref, bat_ref, ps_ref, cn_ref):
    i = pl.program_id(0)
    hn = _bn_relu(t_ref, st_ref, gam_ref, bet_ref)
    oh = (bat_ref[...] == lax.broadcasted_iota(jnp.int32, (NB, NGRP), 1))
    oh = oh.astype(jnp.float32)
    ps = lax.dot_general(oh, hn, (((0,), (0,)), ((), ())),
                         preferred_element_type=jnp.float32)
    cn = lax.dot_general(oh, jnp.ones((NB, 1), jnp.float32),
                         (((0,), (0,)), ((), ())),
                         preferred_element_type=jnp.float32)

    @pl.when(i == 0)
    def _():
        ps_ref[...] = ps
        cn_ref[...] = cn

    @pl.when(i > 0)
    def _():
        ps_ref[...] = ps_ref[...] + ps
        cn_ref[...] = cn_ref[...] + cn


def _tk_pool(t, st, gam, bet, batp):
    return pl.pallas_call(
        _tk_pool_body,
        grid=(GRID,),
        in_specs=[
            pl.BlockSpec((NB, 256), lambda i: (i, 0)),
            _full((2, 256)),
            _full((1, 256)),
            _full((1, 256)),
            pl.BlockSpec((NB, 1), lambda i: (i, 0)),
        ],
        out_specs=[
            _full((NGRP, 256)),
            _full((NGRP, 1)),
        ],
        out_shape=[
            jax.ShapeDtypeStruct((NGRP, 256), jnp.float32),
            jax.ShapeDtypeStruct((NGRP, 1), jnp.float32),
        ],
    )(t, st, gam, bet, batp)


def _tk_head_body(ps_ref, cn_ref, w1_ref, b1_ref, w2_ref, b2_ref,
                  w3_ref, b3_ref, o_ref):
    pooled = ps_ref[...] / jnp.maximum(cn_ref[...], 1.0)
    o = jnp.maximum(pooled, 0.0)
    o = jnp.dot(o, w1_ref[...], preferred_element_type=jnp.float32) + b1_ref[...]
    o = jnp.maximum(o, 0.0)
    o = jnp.dot(o, w2_ref[...], preferred_element_type=jnp.float32) + b2_ref[...]
    o = jnp.maximum(o, 0.0)
    o_ref[...] = jnp.dot(o, w3_ref[...], preferred_element_type=jnp.float32) + b3_ref[...]


def _tk_head(ps, cn, W1, b1, W2, b2, W3, b3):
    return pl.pallas_call(
        _tk_head_body,
        out_shape=jax.ShapeDtypeStruct((NGRP, 10), jnp.float32),
    )(ps, cn, W1, b1, W2, b2, W3, b3)


def kernel(x, edge_index, edge_attr, batch, W_in, b_in, W_gcn, b_gcn,
           bn_gamma, bn_beta, W1, b1, W2, b2, W3, b3):
    row = edge_index[0].astype(jnp.int32)
    col = edge_index[1].astype(jnp.int32)
    ew = edge_attr.astype(jnp.float32)

    npad = E_PAD - E
    spread = (jnp.arange(npad, dtype=jnp.int32) * 37) % N
    row_p = jnp.concatenate([row, spread])
    col_p = jnp.concatenate([col, spread])
    ew_p = jnp.concatenate([ew, jnp.zeros((npad,), jnp.float32)])

    col_d = col_p.reshape(32, NCH_DEG, CHUNK)
    ew_d = ew_p.reshape(32, NCH_DEG, CHUNK)
    eidx = jnp.stack([row_p.reshape(16, NCH, CHUNK),
                      (row_p + N_PAD).reshape(16, NCH, CHUNK),
                      col_p.reshape(16, NCH, CHUNK)], axis=2)
    ew_s = ew_p.reshape(16, NCH, CHUNK)

    degp = _sc_deg(col_d, ew_d)
    degp3 = degp.reshape(2, N_PAD, 1)

    xp = jnp.pad(x, ((0, N_PAD - N), (0, 0)))
    batp = jnp.pad(batch.astype(jnp.int32), (0, N_PAD - N),
                   constant_values=NGRP).reshape(N_PAD, 1)

    h, dis, y2 = _tk_in(xp, W_in, b_in.reshape(1, 256), W_gcn[0], degp3)

    ps = cn = None
    for i in range(4):
        z2 = _sc_spmm(y2.reshape(2 * N_PAD, HH), eidx, ew_s)
        t, st = _tk_stats(z2, y2, h, dis, b_gcn[i].reshape(1, 256))
        gam = bn_gamma[i].reshape(1, 256)
        bet = bn_beta[i].reshape(1, 256)
        if i < 3:
            h, y2 = _tk_norm(t, st, gam, bet, dis, W_gcn[i + 1])
        else:
            ps, cn = _tk_pool(t, st, gam, bet, batp)

    return _tk_head(ps, cn, W1, b1.reshape(1, 128), W2, b2.reshape(1, 64),
                    W3, b3.reshape(1, 10))


# speedup vs baseline: 16.1119x; 1.0224x over previous
import functools

import jax
import jax.numpy as jnp
from jax import lax
from jax.experimental import pallas as pl
from jax.experimental.pallas import tpu as pltpu
from jax.experimental.pallas import tpu_sc as plsc

N = 10000
N_PAD = 10240
NB = 640
GRID = N_PAD // NB
E = 320000
E_PAD = 323584
CHUNK = 128
NCH = E_PAD // 16 // CHUNK
NCH_DEG = E_PAD // 32 // CHUNK
HH = 128
NGRP = 64

_MESH = plsc.VectorSubcoreMesh(core_axis_name="c", subcore_axis_name="s")


@functools.partial(
    pl.kernel,
    out_type=jax.ShapeDtypeStruct((2, N_PAD), jnp.float32),
    mesh=_MESH,
    scratch_types=[
        pltpu.VMEM((NCH_DEG, CHUNK), jnp.int32),
        pltpu.VMEM((NCH_DEG, CHUNK), jnp.float32),
        pltpu.VMEM((NB,), jnp.float32),
        pltpu.VMEM_SHARED((N_PAD,), jnp.float32),
    ],
)
def _sc_deg(col_hbm, ew_hbm, degp_hbm, colv, ewv, zv, acc):
    c = lax.axis_index("c")
    s = lax.axis_index("s")
    wid = c * 16 + s
    for j in range(NB // 16):
        zv[pl.ds(j * 16, 16)] = jnp.zeros((16,), jnp.float32)
    pltpu.sync_copy(zv, acc.at[pl.ds(s * NB, NB)])
    plsc.subcore_barrier()
    pltpu.sync_copy(col_hbm.at[wid], colv)
    pltpu.sync_copy(ew_hbm.at[wid], ewv)

    def body(k, carry):
        pltpu.sync_copy(ewv.at[k], acc.at[colv.at[k]], add=True)
        return carry

    lax.fori_loop(0, NCH_DEG, body, 0)
    plsc.subcore_barrier()
    pltpu.sync_copy(acc.at[pl.ds(s * NB, NB)], degp_hbm.at[c, pl.ds(s * NB, NB)])


@functools.partial(
    pl.kernel,
    out_type=jax.ShapeDtypeStruct((2, N_PAD, HH), jnp.float32),
    mesh=_MESH,
    scratch_types=[
        pltpu.VMEM((4, 2, CHUNK), jnp.int32),
        pltpu.VMEM((4, 2, CHUNK // 2), jnp.int32),
        pltpu.VMEM((4, CHUNK), jnp.float32),
        pltpu.VMEM((2, CHUNK, HH), jnp.float32),
        pltpu.SemaphoreType.DMA,
        pltpu.SemaphoreType.DMA,
        pltpu.SemaphoreType.DMA,
        pltpu.SemaphoreType.DMA,
        pltpu.SemaphoreType.DMA,
        pltpu.VMEM_SHARED((N_PAD, HH), jnp.float32),
    ],
)
def _sc_spmm(yflat_hbm, eidx_hbm, colh_hbm, ew_hbm, z2_hbm, idxb, colb, ewb,
             gbuf, sem0, sem1, ssem0, ssem1, stsem, acc):
    c = lax.axis_index("c")
    s = lax.axis_index("s")
    sems = (sem0, sem1)
    ssems = (ssem0, ssem1)

    def stage_start(k):
        r = k % 4
        pltpu.async_copy(eidx_hbm.at[s, k], idxb.at[r], stsem)
        pltpu.async_copy(colh_hbm.at[s, k], colb.at[r], stsem)
        pltpu.async_copy(ew_hbm.at[s, k], ewb.at[r], stsem)

    def stage_wait(k):
        r = k % 4
        pltpu.make_async_copy(eidx_hbm.at[s, k], idxb.at[r], stsem).wait()
        pltpu.make_async_copy(colh_hbm.at[s, k], colb.at[r], stsem).wait()
        pltpu.make_async_copy(ew_hbm.at[s, k], ewb.at[r], stsem).wait()

    def gather_start(k, slot):
        pltpu.async_copy(yflat_hbm.at[idxb.at[k % 4, c]], gbuf.at[slot],
                         sems[slot])

    def gather_wait(k, slot):
        pltpu.make_async_copy(yflat_hbm.at[idxb.at[k % 4, c]], gbuf.at[slot],
                              sems[slot]).wait()

    def scale_half(k, slot, h):
        r = k % 4
        base = h * (CHUNK // 2)

        def inner(g, carry):
            wv = ewb[r, pl.ds(base + g * 16, 16)]
            for l in range(16):
                e = base + g * 16 + l
                w = wv[l]
                for j in range(HH // 16):
                    sl = pl.ds(j * 16, 16)
                    gbuf[slot, e, sl] = gbuf[slot, e, sl] * w
            return carry

        lax.fori_loop(0, CHUNK // 32, inner, 0)

    def scatter_start_half(k, slot, h):
        pltpu.async_copy(gbuf.at[slot, pl.ds(h * (CHUNK // 2), CHUNK // 2)],
                         acc.at[colb.at[k % 4, h]], ssems[slot], add=True)

    def scatter_wait(k, slot):
        for h in range(2):
            pltpu.make_async_copy(
                gbuf.at[slot, pl.ds(h * (CHUNK // 2), CHUNK // 2)],
                acc.at[colb.at[k % 4, h]], ssems[slot]).wait()

    def zrow(r, carry):
        for j in range(HH // 16):
            gbuf[0, r, pl.ds(j * 16, 16)] = jnp.zeros((16,), jnp.float32)
        return carry

    lax.fori_loop(0, CHUNK, zrow, 0)
    for m in range(NB // CHUNK):
        pltpu.sync_copy(gbuf.at[0], acc.at[pl.ds(s * NB + m * CHUNK, CHUNK)])
    plsc.subcore_barrier()

    stage_start(0)
    stage_start(1)
    stage_start(2)
    stage_wait(0)
    gather_start(0, 0)

    def phase(k, slot, oslot):
        @pl.when(k >= 1)
        def _():
            scatter_wait(k - 1, oslot)

        @pl.when(k + 1 < NCH)
        def _():
            stage_wait(k + 1)
            gather_start(k + 1, oslot)

        gather_wait(k, slot)
        scale_half(k, slot, 0)
        scatter_start_half(k, slot, 0)
        scale_half(k, slot, 1)
        scatter_start_half(k, slot, 1)

        @pl.when(k + 3 < NCH)
        def _():
            stage_start(k + 3)

    def body(k, carry):
        even = (k % 2) == 0

        @pl.when(even)
        def _():
            phase(k, 0, 1)

        @pl.when(jnp.logical_not(even))
        def _():
            phase(k, 1, 0)

        return carry

    lax.fori_loop(0, NCH, body, 0)
    scatter_wait(NCH - 1, (NCH - 1) % 2)
    plsc.subcore_barrier()
    pltpu.sync_copy(acc.at[pl.ds(s * NB, NB)], z2_hbm.at[c, pl.ds(s * NB, NB)])


def _full(shape):
    return pl.BlockSpec(shape, lambda i: tuple(0 for _ in shape))


def _tk_in_body(x_ref, win_ref, bin_ref, w0_ref, degp_ref,
                h_ref, dis_ref, y2_ref):
    h = jnp.dot(x_ref[...], win_ref[...], preferred_element_type=jnp.float32)
    h = h + bin_ref[...]
    dis = lax.rsqrt(1.0 + degp_ref[0] + degp_ref[1])
    h_ref[...] = h
    dis_ref[...] = dis
    y = dis * jnp.dot(h, w0_ref[...], preferred_element_type=jnp.float32)
    y2_ref[0] = y[:, :HH]
    y2_ref[1] = y[:, HH:]


def _tk_in(xp, W_in, b_in, W0, degp3):
    return pl.pallas_call(
        _tk_in_body,
        grid=(GRID,),
        in_specs=[
            pl.BlockSpec((NB, 128), lambda i: (i, 0)),
            _full((128, 256)),
            _full((1, 256)),
            _full((256, 256)),
            pl.BlockSpec((2, NB, 1), lambda i: (0, i, 0)),
        ],
        out_specs=[
            pl.BlockSpec((NB, 256), lambda i: (i, 0)),
            pl.BlockSpec((NB, 1), lambda i: (i, 0)),
            pl.BlockSpec((2, NB, HH), lambda i: (0, i, 0)),
        ],
        out_shape=[
            jax.ShapeDtypeStruct((N_PAD, 256), jnp.float32),
            jax.ShapeDtypeStruct((N_PAD, 1), jnp.float32),
            jax.ShapeDtypeStruct((2, N_PAD, HH), jnp.float32),
        ],
    )(xp, W_in, b_in, W0, degp3)


def _tk_stats_body(z2_ref, y2_ref, h_ref, dis_ref, b_ref, t_ref, st_ref):
    i = pl.program_id(0)
    z = jnp.concatenate([z2_ref[0], z2_ref[1]], axis=1)
    y = jnp.concatenate([y2_ref[0], y2_ref[1]], axis=1)
    t = dis_ref[...] * (z + y) + b_ref[...] + h_ref[...]
    t_ref[...] = t
    rows = i * NB + lax.broadcasted_iota(jnp.int32, (NB, 1), 0)
    tm = jnp.where(rows < N, t, 0.0)
    s1 = jnp.sum(tm, axis=0, keepdims=True)
    s2 = jnp.sum(tm * tm, axis=0, keepdims=True)
    blk = jnp.concatenate([s1, s2], axis=0)

    @pl.when(i == 0)
    def _():
        st_ref[...] = blk

    @pl.when(i > 0)
    def _():
        st_ref[...] = st_ref[...] + blk


def _tk_stats(z2, y2, h, dis, b):
    return pl.pallas_call(
        _tk_stats_body,
        grid=(GRID,),
        in_specs=[
            pl.BlockSpec((2, NB, HH), lambda i: (0, i, 0)),
            pl.BlockSpec((2, NB, HH), lambda i: (0, i, 0)),
            pl.BlockSpec((NB, 256), lambda i: (i, 0)),
            pl.BlockSpec((NB, 1), lambda i: (i, 0)),
            _full((1, 256)),
        ],
        out_specs=[
            pl.BlockSpec((NB, 256), lambda i: (i, 0)),
            _full((2, 256)),
        ],
        out_shape=[
            jax.ShapeDtypeStruct((N_PAD, 256), jnp.float32),
            jax.ShapeDtypeStruct((2, 256), jnp.float32),
        ],
    )(z2, y2, h, dis, b)


def _bn_relu(t_ref, st_ref, gam_ref, bet_ref):
    mu = st_ref[0:1, :] * (1.0 / N)
    var = st_ref[1:2, :] * (1.0 / N) - mu * mu
    inv = lax.rsqrt(var + 1e-5)
    return jnp.maximum(gam_ref[...] * (t_ref[...] - mu) * inv + bet_ref[...], 0.0)


def _tk_norm_body(t_ref, st_ref, gam_ref, bet_ref, dis_ref, w_ref,
                  h_ref, y2_ref):
    hn = _bn_relu(t_ref, st_ref, gam_ref, bet_ref)
    h_ref[...] = hn
    y = dis_ref[...] * jnp.dot(hn, w_ref[...], preferred_element_type=jnp.float32)
    y2_ref[0] = y[:, :HH]
    y2_ref[1] = y[:, HH:]


def _tk_norm(t, st, gam, bet, dis, Wn):
    return pl.pallas_call(
        _tk_norm_body,
        grid=(GRID,),
        in_specs=[
            pl.BlockSpec((NB, 256), lambda i: (i, 0)),
            _full((2, 256)),
            _full((1, 256)),
            _full((1, 256)),
            pl.BlockSpec((NB, 1), lambda i: (i, 0)),
            _full((256, 256)),
        ],
        out_specs=[
            pl.BlockSpec((NB, 256), lambda i: (i, 0)),
            pl.BlockSpec((2, NB, HH), lambda i: (0, i, 0)),
        ],
        out_shape=[
            jax.ShapeDtypeStruct((N_PAD, 256), jnp.float32),
            jax.ShapeDtypeStruct((2, N_PAD, HH), jnp.float32),
        ],
    )(t, st, gam, bet, dis, Wn)


def _tk_pool_body(t_ref, st_ref, gam_ref, bet_ref, bat_ref, ps_ref, cn_ref):
    i = pl.program_id(0)
    hn = _bn_relu(t_ref, st_ref, gam_ref, bet_ref)
    oh = (bat_ref[...] == lax.broadcasted_iota(jnp.int32, (NB, NGRP), 1))
    oh = oh.astype(jnp.float32)
    ps = lax.dot_general(oh, hn, (((0,), (0,)), ((), ())),
                         preferred_element_type=jnp.float32)
    cn = lax.dot_general(oh, jnp.ones((NB, 1), jnp.float32),
                         (((0,), (0,)), ((), ())),
                         preferred_element_type=jnp.float32)

    @pl.when(i == 0)
    def _():
        ps_ref[...] = ps
        cn_ref[...] = cn

    @pl.when(i > 0)
    def _():
        ps_ref[...] = ps_ref[...] + ps
        cn_ref[...] = cn_ref[...] + cn


def _tk_pool(t, st, gam, bet, batp):
    return pl.pallas_call(
        _tk_pool_body,
        grid=(GRID,),
        in_specs=[
            pl.BlockSpec((NB, 256), lambda i: (i, 0)),
            _full((2, 256)),
            _full((1, 256)),
            _full((1, 256)),
            pl.BlockSpec((NB, 1), lambda i: (i, 0)),
        ],
        out_specs=[
            _full((NGRP, 256)),
            _full((NGRP, 1)),
        ],
        out_shape=[
            jax.ShapeDtypeStruct((NGRP, 256), jnp.float32),
            jax.ShapeDtypeStruct((NGRP, 1), jnp.float32),
        ],
    )(t, st, gam, bet, batp)


def _tk_head_body(ps_ref, cn_ref, w1_ref, b1_ref, w2_ref, b2_ref,
                  w3_ref, b3_ref, o_ref):
    pooled = ps_ref[...] / jnp.maximum(cn_ref[...], 1.0)
    o = jnp.maximum(pooled, 0.0)
    o = jnp.dot(o, w1_ref[...], preferred_element_type=jnp.float32) + b1_ref[...]
    o = jnp.maximum(o, 0.0)
    o = jnp.dot(o, w2_ref[...], preferred_element_type=jnp.float32) + b2_ref[...]
    o = jnp.maximum(o, 0.0)
    o_ref[...] = jnp.dot(o, w3_ref[...], preferred_element_type=jnp.float32) + b3_ref[...]


def _tk_head(ps, cn, W1, b1, W2, b2, W3, b3):
    return pl.pallas_call(
        _tk_head_body,
        out_shape=jax.ShapeDtypeStruct((NGRP, 10), jnp.float32),
    )(ps, cn, W1, b1, W2, b2, W3, b3)


def kernel(x, edge_index, edge_attr, batch, W_in, b_in, W_gcn, b_gcn,
           bn_gamma, bn_beta, W1, b1, W2, b2, W3, b3):
    row = edge_index[0].astype(jnp.int32)
    col = edge_index[1].astype(jnp.int32)
    ew = edge_attr.astype(jnp.float32)

    npad = E_PAD - E
    spread = (jnp.arange(npad, dtype=jnp.int32) * 37) % N
    row_p = jnp.concatenate([row, spread])
    col_p = jnp.concatenate([col, spread])
    ew_p = jnp.concatenate([ew, jnp.zeros((npad,), jnp.float32)])

    col_d = col_p.reshape(32, NCH_DEG, CHUNK)
    ew_d = ew_p.reshape(32, NCH_DEG, CHUNK)
    eidx = jnp.stack([row_p.reshape(16, NCH, CHUNK),
                      (row_p + N_PAD).reshape(16, NCH, CHUNK)], axis=2)
    colh = col_p.reshape(16, NCH, 2, CHUNK // 2)
    ew_s = ew_p.reshape(16, NCH, CHUNK)

    degp = _sc_deg(col_d, ew_d)
    degp3 = degp.reshape(2, N_PAD, 1)

    xp = jnp.pad(x, ((0, N_PAD - N), (0, 0)))
    batp = jnp.pad(batch.astype(jnp.int32), (0, N_PAD - N),
                   constant_values=NGRP).reshape(N_PAD, 1)

    h, dis, y2 = _tk_in(xp, W_in, b_in.reshape(1, 256), W_gcn[0], degp3)

    ps = cn = None
    for i in range(4):
        z2 = _sc_spmm(y2.reshape(2 * N_PAD, HH), eidx, colh, ew_s)
        t, st = _tk_stats(z2, y2, h, dis, b_gcn[i].reshape(1, 256))
        gam = bn_gamma[i].reshape(1, 256)
        bet = bn_beta[i].reshape(1, 256)
        if i < 3:
            h, y2 = _tk_norm(t, st, gam, bet, dis, W_gcn[i + 1])
        else:
            ps, cn = _tk_pool(t, st, gam, bet, batp)

    return _tk_head(ps, cn, W1, b1.reshape(1, 128), W2, b2.reshape(1, 64),
                    W3, b3.reshape(1, 10))
